# baseline (speedup 1.0000x reference)
import jax
import jax.numpy as jnp
from jax import lax
from jax.experimental import pallas as pl
from jax.experimental.pallas import tpu as pltpu
from jax.experimental.pallas import tpu_sc as plsc

NHID = 128
QUEUE_SIZE = 16384
NQUEUE = 16
NROWS = NQUEUE * NHID
CR = 64
NCH = NROWS // CR
NB = 8
D = 4
RPQ = NHID // CR
NIDS = NQUEUE * QUEUE_SIZE
NWORK = 32
IPW = NIDS // NWORK


def _in_cp(q_hbm, buf, in_sems, ch, b):
    return pltpu.make_async_copy(
        q_hbm.at[pl.ds(ch * CR, CR), :], buf.at[b], in_sems.at[b])


def _out_cp(q_out, buf, out_sems, ch, b):
    return pltpu.make_async_copy(
        buf.at[b], q_out.at[pl.ds(ch * CR, CR), :], out_sems.at[b])


def _tc_body(qid_ref, qptr_smem, k_ref, q_hbm, q_out, buf, in_sems, out_sems):
    qid = qid_ref[0]
    ptr = qptr_smem[qid]

    for ch in range(D):
        _in_cp(q_hbm, buf, in_sems, ch, ch % NB).start()

    lane2 = jax.lax.broadcasted_iota(jnp.int32, (CR, QUEUE_SIZE), 1)

    def super_step(s, carry):
        for b in range(NB):
            ch = s * NB + b
            _in_cp(q_hbm, buf, in_sems, ch, b).wait()

            hit = (ch >= qid * RPQ) & (ch < qid * RPQ + RPQ)

            @pl.when(hit)
            def _patch(ch=ch, b=b):
                off = ch * CR - qid * NHID
                kb = k_ref[pl.ds(off, CR), :]
                buf[b] = jnp.where(lane2 == ptr, kb, buf[b])

            _out_cp(q_out, buf, out_sems, ch, b).start()

            pf = ch + D
            bp = (b + D) % NB

            @pl.when(pf < NCH)
            def _prefetch(pf=pf, bp=bp):
                @pl.when(pf >= NB)
                def _drain(pf=pf, bp=bp):
                    _out_cp(q_out, buf, out_sems, pf - NB, bp).wait()
                _in_cp(q_hbm, buf, in_sems, pf, bp).start()
        return carry

    jax.lax.fori_loop(0, NCH // NB, super_step, 0)

    for j in range(NB):
        ch = NCH - NB + j
        _out_cp(q_out, buf, out_sems, ch, ch % NB).wait()


def _sc_body(ids_hbm, qptr_hbm, qid_hbm, elem_hbm, ids_out, ptr_out,
             buf, qv, qidv, elemv, scat_sem):
    c = lax.axis_index("c")
    s = lax.axis_index("s")
    wid = s * 2 + c
    base = wid * IPW

    pltpu.sync_copy(ids_hbm.at[pl.ds(base, IPW)], buf)
    pltpu.sync_copy(qptr_hbm, qv)
    pltpu.sync_copy(qid_hbm, qidv)
    pltpu.sync_copy(elem_hbm, elemv)
    pltpu.sync_copy(buf, ids_out.at[pl.ds(base, IPW)])

    plsc.subcore_barrier()

    @pl.when(wid == 0)
    def _scatter():
        qid_vec = qidv[...]
        qptr_vec = qv[...]
        lane = lax.iota(jnp.int32, 16)
        ptr_b = lax.gather(
            qptr_vec, qid_vec[:, None],
            lax.GatherDimensionNumbers(offset_dims=(),
                                       collapsed_slice_dims=(0,),
                                       start_index_map=(0,)),
            slice_sizes=(1,),
            mode=lax.GatherScatterMode.PROMISE_IN_BOUNDS)
        tgt = qid_vec * QUEUE_SIZE + ptr_b
        pltpu.async_copy(elemv, ids_out.at[tgt], scat_sem).wait()

        newptr = jnp.where(lane == qid_vec, (ptr_b + 1) % QUEUE_SIZE, qptr_vec)
        qv[...] = newptr
        pltpu.sync_copy(qv, ptr_out)


def kernel(k, queue, ids, queue_ptr, elem_id, q_id):
    qid = jnp.asarray(q_id, jnp.int32).reshape(1)
    qid16 = jnp.full((16,), q_id, jnp.int32)
    elem16 = jnp.full((16,), elem_id, jnp.int32)
    k2 = k.reshape(NHID, 1)
    q2 = queue.reshape(NROWS, QUEUE_SIZE)
    ids1 = ids.reshape(NIDS)

    out_q = pl.pallas_call(
        _tc_body,
        in_specs=[
            pl.BlockSpec(memory_space=pltpu.SMEM),
            pl.BlockSpec(memory_space=pltpu.SMEM),
            pl.BlockSpec(memory_space=pltpu.VMEM),
            pl.BlockSpec(memory_space=pl.ANY),
        ],
        out_specs=pl.BlockSpec(memory_space=pl.ANY),
        out_shape=jax.ShapeDtypeStruct((NROWS, QUEUE_SIZE), jnp.float32),
        scratch_shapes=[
            pltpu.VMEM((NB, CR, QUEUE_SIZE), jnp.float32),
            pltpu.SemaphoreType.DMA((NB,)),
            pltpu.SemaphoreType.DMA((NB,)),
        ],
    )(qid, queue_ptr, k2, q2)

    mesh = plsc.VectorSubcoreMesh(core_axis_name="c", subcore_axis_name="s")
    out_ids, out_ptr = pl.kernel(
        _sc_body,
        out_type=[
            jax.ShapeDtypeStruct((NIDS,), jnp.int32),
            jax.ShapeDtypeStruct((NQUEUE,), jnp.int32),
        ],
        mesh=mesh,
        scratch_types=[
            pltpu.VMEM((IPW,), jnp.int32),
            pltpu.VMEM((NQUEUE,), jnp.int32),
            pltpu.VMEM((16,), jnp.int32),
            pltpu.VMEM((16,), jnp.int32),
            pltpu.SemaphoreType.DMA,
        ],
    )(ids1, queue_ptr, qid16, elem16)

    return (out_q.reshape(NQUEUE, NHID, QUEUE_SIZE),
            out_ids.reshape(NQUEUE, QUEUE_SIZE), out_ptr)


# speedup vs baseline: 1.0009x; 1.0009x over previous
import jax
import jax.numpy as jnp
from jax import lax
from jax.experimental import pallas as pl
from jax.experimental.pallas import tpu as pltpu
from jax.experimental.pallas import tpu_sc as plsc

NHID = 128
QUEUE_SIZE = 16384
NQUEUE = 16
NROWS = NQUEUE * NHID
CR = 64
NCH = NROWS // CR
NB = 8
D = 4
RPQ = NHID // CR
NIDS = NQUEUE * QUEUE_SIZE
NWORK = 32
IPW = NIDS // NWORK


def _in_cp(q_hbm, buf, in_sems, ch, b):
    return pltpu.make_async_copy(
        q_hbm.at[pl.ds(ch * CR, CR), :], buf.at[b], in_sems.at[b])


def _out_cp(q_out, buf, out_sems, ch, b):
    return pltpu.make_async_copy(
        buf.at[b], q_out.at[pl.ds(ch * CR, CR), :], out_sems.at[b])


def _tc_body(qid_ref, qptr_smem, k_ref, q_hbm, q_out, buf, in_sems, out_sems):
    qid = qid_ref[0]
    ptr = qptr_smem[qid]

    for ch in range(D):
        _in_cp(q_hbm, buf, in_sems, ch, ch % NB).start()

    lane2 = jax.lax.broadcasted_iota(jnp.int32, (CR, QUEUE_SIZE), 1)

    def super_step(s, carry):
        for b in range(NB):
            ch = s * NB + b
            _in_cp(q_hbm, buf, in_sems, ch, b).wait()

            hit = (ch >= qid * RPQ) & (ch < qid * RPQ + RPQ)

            @pl.when(hit)
            def _patch(ch=ch, b=b):
                off = ch * CR - qid * NHID
                kb = k_ref[pl.ds(off, CR), :]
                buf[b] = jnp.where(lane2 == ptr, kb, buf[b])

            _out_cp(q_out, buf, out_sems, ch, b).start()

            pf = ch + D
            bp = (b + D) % NB

            @pl.when(pf < NCH)
            def _prefetch(pf=pf, bp=bp):
                @pl.when(pf >= NB)
                def _drain(pf=pf, bp=bp):
                    _out_cp(q_out, buf, out_sems, pf - NB, bp).wait()
                _in_cp(q_hbm, buf, in_sems, pf, bp).start()
        return carry

    jax.lax.fori_loop(0, NCH // NB, super_step, 0)

    for j in range(NB):
        ch = NCH - NB + j
        _out_cp(q_out, buf, out_sems, ch, ch % NB).wait()


def _sc_body(ids_hbm, qptr_hbm, qid_hbm, elem_hbm, ids_out, ptr_out,
             buf, qv, qidv, elemv, scat_sem):
    c = lax.axis_index("c")
    s = lax.axis_index("s")
    wid = s * 2 + c
    base = wid * IPW

    pltpu.sync_copy(ids_hbm.at[pl.ds(base, IPW)], buf)
    pltpu.sync_copy(qptr_hbm, qv)
    pltpu.sync_copy(qid_hbm, qidv)
    pltpu.sync_copy(elem_hbm, elemv)
    pltpu.sync_copy(buf, ids_out.at[pl.ds(base, IPW)])

    plsc.subcore_barrier()

    @pl.when(wid == 0)
    def _scatter():
        qid_vec = qidv[...]
        qptr_vec = qv[...]
        lane = lax.iota(jnp.int32, 16)
        ptr_b = lax.gather(
            qptr_vec, qid_vec[:, None],
            lax.GatherDimensionNumbers(offset_dims=(),
                                       collapsed_slice_dims=(0,),
                                       start_index_map=(0,)),
            slice_sizes=(1,),
            mode=lax.GatherScatterMode.PROMISE_IN_BOUNDS)
        tgt = qid_vec * QUEUE_SIZE + ptr_b
        pltpu.async_copy(elemv, ids_out.at[tgt], scat_sem).wait()

        newptr = jnp.where(lane == qid_vec, (ptr_b + 1) % QUEUE_SIZE, qptr_vec)
        qv[...] = newptr
        pltpu.sync_copy(qv, ptr_out)


def kernel(k, queue, ids, queue_ptr, elem_id, q_id):
    qid = jnp.asarray(q_id, jnp.int32).reshape(1)
    qid16 = jnp.full((16,), q_id, jnp.int32)
    elem16 = jnp.full((16,), elem_id, jnp.int32)
    k2 = k.reshape(NHID, 1)
    q2 = queue.reshape(NROWS, QUEUE_SIZE)
    ids1 = ids.reshape(NIDS)

    mesh = plsc.VectorSubcoreMesh(core_axis_name="c", subcore_axis_name="s")
    out_ids, out_ptr = pl.kernel(
        _sc_body,
        out_type=[
            jax.ShapeDtypeStruct((NIDS,), jnp.int32),
            jax.ShapeDtypeStruct((NQUEUE,), jnp.int32),
        ],
        mesh=mesh,
        scratch_types=[
            pltpu.VMEM((IPW,), jnp.int32),
            pltpu.VMEM((NQUEUE,), jnp.int32),
            pltpu.VMEM((16,), jnp.int32),
            pltpu.VMEM((16,), jnp.int32),
            pltpu.SemaphoreType.DMA,
        ],
    )(ids1, queue_ptr, qid16, elem16)

    out_q = pl.pallas_call(
        _tc_body,
        in_specs=[
            pl.BlockSpec(memory_space=pltpu.SMEM),
            pl.BlockSpec(memory_space=pltpu.SMEM),
            pl.BlockSpec(memory_space=pltpu.VMEM),
            pl.BlockSpec(memory_space=pl.ANY),
        ],
        out_specs=pl.BlockSpec(memory_space=pl.ANY),
        out_shape=jax.ShapeDtypeStruct((NROWS, QUEUE_SIZE), jnp.float32),
        scratch_shapes=[
            pltpu.VMEM((NB, CR, QUEUE_SIZE), jnp.float32),
            pltpu.SemaphoreType.DMA((NB,)),
            pltpu.SemaphoreType.DMA((NB,)),
        ],
    )(qid, queue_ptr, k2, q2)

    return (out_q.reshape(NQUEUE, NHID, QUEUE_SIZE),
            out_ids.reshape(NQUEUE, QUEUE_SIZE), out_ptr)


# speedup vs baseline: 1.2234x; 1.2224x over previous
import jax
import jax.numpy as jnp
from jax.experimental import pallas as pl
from jax.experimental.pallas import tpu as pltpu

NHID = 128
QUEUE_SIZE = 16384
NQUEUE = 16
NROWS = NQUEUE * NHID
CR = 16
NCH = NROWS // CR
NB = 16
D = 8
RPQ = NHID // CR


def _in_cp(q_hbm, buf, in_sems, ch, b):
    return pltpu.make_async_copy(
        q_hbm.at[pl.ds(ch * CR, CR), :], buf.at[b], in_sems.at[b])


def _out_cp(q_out, buf, out_sems, ch, b):
    return pltpu.make_async_copy(
        buf.at[b], q_out.at[pl.ds(ch * CR, CR), :], out_sems.at[b])


def _body(qid_ref, elem_ref, qptr_smem, k_ref, q_hbm, ids_hbm, qptr_v,
          q_out, ids_out, ptr_out, buf, ids_buf, in_sems, out_sems, ids_sem):
    qid = qid_ref[0]
    ptr = qptr_smem[qid]

    pltpu.make_async_copy(ids_hbm, ids_buf, ids_sem).start()
    for ch in range(D):
        _in_cp(q_hbm, buf, in_sems, ch, ch % NB).start()

    lane2 = jax.lax.broadcasted_iota(jnp.int32, (CR, QUEUE_SIZE), 1)

    def super_step(s, carry):
        for b in range(NB):
            ch = s * NB + b
            _in_cp(q_hbm, buf, in_sems, ch, b).wait()

            hit = (ch >= qid * RPQ) & (ch < qid * RPQ + RPQ)

            @pl.when(hit)
            def _patch(ch=ch, b=b):
                off = ch * CR - qid * NHID
                kb = k_ref[pl.ds(off, CR), :]
                buf[b] = jnp.where(lane2 == ptr, kb, buf[b])

            _out_cp(q_out, buf, out_sems, ch, b).start()

            pf = ch + D
            bp = (b + D) % NB

            @pl.when(pf < NCH)
            def _prefetch(pf=pf, bp=bp):
                @pl.when(pf >= NB)
                def _drain(pf=pf, bp=bp):
                    _out_cp(q_out, buf, out_sems, pf - NB, bp).wait()
                _in_cp(q_hbm, buf, in_sems, pf, bp).start()
        return carry

    jax.lax.fori_loop(0, NCH // NB, super_step, 0)

    pltpu.make_async_copy(ids_hbm, ids_buf, ids_sem).wait()
    lane_ids = jax.lax.broadcasted_iota(jnp.int32, (NQUEUE, QUEUE_SIZE), 1)
    row_ids = jax.lax.broadcasted_iota(jnp.int32, (NQUEUE, QUEUE_SIZE), 0)
    ids_buf[...] = jnp.where((lane_ids == ptr) & (row_ids == qid),
                             elem_ref[0], ids_buf[...])
    pltpu.make_async_copy(ids_buf, ids_out, ids_sem).start()

    lane = jax.lax.broadcasted_iota(jnp.int32, (1, NQUEUE), 1)
    ptr_out[...] = jnp.where(lane == qid, (ptr + 1) % QUEUE_SIZE, qptr_v[...])

    for j in range(NB):
        ch = NCH - NB + j
        _out_cp(q_out, buf, out_sems, ch, ch % NB).wait()
    pltpu.make_async_copy(ids_buf, ids_out, ids_sem).wait()


def kernel(k, queue, ids, queue_ptr, elem_id, q_id):
    qid = jnp.asarray(q_id, jnp.int32).reshape(1)
    elem = jnp.asarray(elem_id, jnp.int32).reshape(1)
    k2 = k.reshape(NHID, 1)
    q2 = queue.reshape(NROWS, QUEUE_SIZE)
    qptr2 = queue_ptr.reshape(1, NQUEUE)

    out_q, out_ids, out_ptr = pl.pallas_call(
        _body,
        in_specs=[
            pl.BlockSpec(memory_space=pltpu.SMEM),
            pl.BlockSpec(memory_space=pltpu.SMEM),
            pl.BlockSpec(memory_space=pltpu.SMEM),
            pl.BlockSpec(memory_space=pltpu.VMEM),
            pl.BlockSpec(memory_space=pl.ANY),
            pl.BlockSpec(memory_space=pl.ANY),
            pl.BlockSpec(memory_space=pltpu.VMEM),
        ],
        out_specs=[
            pl.BlockSpec(memory_space=pl.ANY),
            pl.BlockSpec(memory_space=pl.ANY),
            pl.BlockSpec(memory_space=pltpu.VMEM),
        ],
        out_shape=[
            jax.ShapeDtypeStruct((NROWS, QUEUE_SIZE), jnp.float32),
            jax.ShapeDtypeStruct((NQUEUE, QUEUE_SIZE), jnp.int32),
            jax.ShapeDtypeStruct((1, NQUEUE), jnp.int32),
        ],
        scratch_shapes=[
            pltpu.VMEM((NB, CR, QUEUE_SIZE), jnp.float32),
            pltpu.VMEM((NQUEUE, QUEUE_SIZE), jnp.int32),
            pltpu.SemaphoreType.DMA((NB,)),
            pltpu.SemaphoreType.DMA((NB,)),
            pltpu.SemaphoreType.DMA,
        ],
    )(qid, elem, queue_ptr, k2, q2, ids, qptr2)
    return (out_q.reshape(NQUEUE, NHID, QUEUE_SIZE), out_ids,
            out_ptr.reshape(NQUEUE))
